# Initial kernel scaffold; baseline (speedup 1.0000x reference)
import jax
import jax.numpy as jnp
from jax.experimental import pallas as pl


def kernel(x, V, batch, W1, b1, W2, b2, W3, b3, W4, b4):
    raise NotImplementedError("write your pallas kernel here")


# speedup vs baseline: 2.8904x; 2.8904x over previous
import functools

import jax
import jax.numpy as jnp
from jax.experimental import pallas as pl
from jax.experimental.pallas import tpu as pltpu

N = 100000
D = 128
NWIDTH = 16
H = 128
G = 512
TILE = 1000
NB = N // TILE


def _body(x_ref, v_ref, b_ref, W1_ref, b1_ref, W2_ref, b2_ref,
          W3_ref, b3_ref, W4_ref, b4_ref, out_ref, acc_ref):
    i = pl.program_id(0)

    @pl.when(i == 0)
    def _init():
        acc_ref[...] = jnp.zeros_like(acc_ref)

    xv = x_ref[...]
    v = v_ref[...]
    inv = jnp.sqrt(v[:, 0:16] ** 2 + v[:, 16:32] ** 2 + v[:, 32:48] ** 2)
    h = (jax.lax.dot(xv, W1_ref[0:D, :], preferred_element_type=jnp.float32)
         + jax.lax.dot(inv, W1_ref[D:D + NWIDTH, :],
                       preferred_element_type=jnp.float32)
         + b1_ref[...])
    h = h * jax.nn.sigmoid(h)
    h = jax.lax.dot(h, W2_ref[...], preferred_element_type=jnp.float32) + b2_ref[...]

    ids = b_ref[0]
    rows = jax.lax.broadcasted_iota(jnp.int32, (G, TILE), 0)
    ohT = (rows == ids).astype(jnp.float32)
    acc_ref[...] += jax.lax.dot(ohT, h, preferred_element_type=jnp.float32)

    @pl.when(i == NB - 1)
    def _post():
        g = jax.lax.dot(acc_ref[...], W3_ref[...],
                        preferred_element_type=jnp.float32) + b3_ref[...]
        g = g * jax.nn.sigmoid(g)
        out_ref[...] = (jax.lax.dot(g, W4_ref[...],
                                    preferred_element_type=jnp.float32)
                        + b4_ref[...])


@jax.jit
def kernel(x, V, batch, W1, b1, W2, b2, W3, b3, W4, b4):
    v2 = V.reshape(N, 3 * NWIDTH)
    b3d = batch.reshape(NB, 1, TILE)
    grid = (NB,)
    full = lambda *s: pl.BlockSpec(s, lambda i: (0,) * len(s))
    out = pl.pallas_call(
        _body,
        grid=grid,
        in_specs=[
            pl.BlockSpec((TILE, D), lambda i: (i, 0)),
            pl.BlockSpec((TILE, 3 * NWIDTH), lambda i: (i, 0)),
            pl.BlockSpec((1, 1, TILE), lambda i: (i, 0, 0)),
            full(D + NWIDTH, H),
            full(1, H),
            full(H, H),
            full(1, H),
            full(H, H),
            full(1, H),
            full(H, 1),
            full(1, 1),
        ],
        out_specs=pl.BlockSpec((G, 1), lambda i: (0, 0)),
        out_shape=jax.ShapeDtypeStruct((G, 1), jnp.float32),
        scratch_shapes=[pltpu.VMEM((G, H), jnp.float32)],
        compiler_params=pltpu.CompilerParams(
            dimension_semantics=("arbitrary",),
        ),
    )(x, v2, b3d, W1, b1.reshape(1, H), W2, b2.reshape(1, H),
      W3, b3.reshape(1, H), W4, b4.reshape(1, 1))
    return out
